# baseline (speedup 1.0000x reference)
import functools

import jax
import jax.numpy as jnp
from jax import lax
from jax.experimental import pallas as pl
from jax.experimental.pallas import tpu as pltpu
from jax.experimental.pallas import tpu_sc as plsc

D_MODEL = 64
SCALE = 8.0
NUM_WORKERS = 32
CHUNK = 256


def kernel(x, table):
    s0, s1 = x.shape
    bsz = s0 * s1
    vocab = table.shape[0]
    xf = x.reshape(bsz).astype(jnp.int32)
    table2 = table.reshape(vocab // 2, 2 * D_MODEL)
    b_per_w = bsz // NUM_WORKERS
    n_chunks = b_per_w // CHUNK
    assert n_chunks % 2 == 0 and n_chunks >= 4

    mesh = plsc.VectorSubcoreMesh(core_axis_name="c", subcore_axis_name="s")

    @functools.partial(
        pl.kernel,
        mesh=mesh,
        out_type=jax.ShapeDtypeStruct((bsz * D_MODEL,), jnp.float32),
        scratch_types=[
            [pltpu.VMEM((CHUNK,), jnp.int32) for _ in range(2)],
            [pltpu.VMEM((CHUNK,), jnp.int32) for _ in range(2)],
            [pltpu.VMEM((CHUNK, 2 * D_MODEL), jnp.float32) for _ in range(2)],
            [pltpu.VMEM((CHUNK * D_MODEL,), jnp.float32) for _ in range(2)],
            [pltpu.SemaphoreType.DMA for _ in range(2)],
            [pltpu.SemaphoreType.DMA for _ in range(2)],
        ],
        compiler_params=pltpu.CompilerParams(needs_layout_passes=False),
    )
    def emb(x_hbm, table_hbm, out_hbm, idx, rowid, gath, outb, gsem, ssem):
        wid = lax.axis_index("s") * 2 + lax.axis_index("c")
        base = wid * b_per_w
        lane = lax.iota(jnp.int32, 16)

        def start_gather(g, b):
            pltpu.sync_copy(x_hbm.at[pl.ds(base + g * CHUNK, CHUNK)], idx[b])
            for i in range(CHUNK // 16):
                sl = pl.ds(i * 16, 16)
                rowid[b][sl] = jax.lax.shift_right_logical(idx[b][sl], 1)
            pltpu.async_copy(table_hbm.at[rowid[b]], gath[b], gsem[b])

        def wait_gather(b):
            pltpu.make_async_copy(
                table_hbm.at[pl.ds(0, CHUNK)], gath[b], gsem[b]
            ).wait()

        def issue_scatter(g, b):
            dst = out_hbm.at[pl.ds((base + g * CHUNK) * D_MODEL, CHUNK * D_MODEL)]
            pltpu.async_copy(outb[b], dst, ssem[b])

        def wait_scatter(b):
            pltpu.make_async_copy(
                outb[b], out_hbm.at[pl.ds(0, CHUNK * D_MODEL)], ssem[b]
            ).wait()

        def turn(g, b, first, last):
            if not first:
                wait_scatter(b)
            wait_gather(b)

            @plsc.parallel_loop(0, CHUNK, step=16)
            def _compact(r0):
                par = (idx[b][pl.ds(r0, 16)] & 1) * D_MODEL
                for j in range(16):
                    r = r0 + j
                    rowv = jax.lax.broadcast(r, (16,))
                    src_half = par[j]
                    for c in range(D_MODEL // 16):
                        colv = src_half + c * 16 + lane
                        vals = plsc.load_gather(gath[b], [rowv, colv])
                        outb[b][pl.ds(r * D_MODEL + c * 16, 16)] = vals * SCALE

            issue_scatter(g, b)
            if not last:
                start_gather(g + 2, b)

        start_gather(0, 0)
        start_gather(1, 1)

        turn(0, 0, True, False)
        turn(1, 1, True, False)

        def cycle(gg, carry):
            turn(2 * gg, 0, False, False)
            turn(2 * gg + 1, 1, False, False)
            return carry

        lax.fori_loop(1, n_chunks // 2 - 1, cycle, 0)

        turn(n_chunks - 2, 0, False, True)
        turn(n_chunks - 1, 1, False, True)

        wait_scatter(0)
        wait_scatter(1)

    out = emb(xf, table2)
    return out.reshape(s0, s1, D_MODEL)


# speedup vs baseline: 1.2052x; 1.2052x over previous
import functools

import jax
import jax.numpy as jnp
from jax import lax
from jax.experimental import pallas as pl
from jax.experimental.pallas import tpu as pltpu
from jax.experimental.pallas import tpu_sc as plsc

D_MODEL = 64
SCALE = 8.0
NUM_WORKERS = 32
CHUNK = 128


def kernel(x, table):
    s0, s1 = x.shape
    bsz = s0 * s1
    vocab = table.shape[0]
    xf = x.reshape(bsz).astype(jnp.int32)
    table2 = table.reshape(vocab // 2, 2 * D_MODEL)
    b_per_w = bsz // NUM_WORKERS
    n_chunks = b_per_w // CHUNK
    assert n_chunks % 2 == 0 and n_chunks >= 4

    mesh = plsc.VectorSubcoreMesh(core_axis_name="c", subcore_axis_name="s")

    @functools.partial(
        pl.kernel,
        mesh=mesh,
        out_type=jax.ShapeDtypeStruct((bsz, D_MODEL), jnp.float32),
        scratch_types=[
            pltpu.VMEM((b_per_w,), jnp.int32),
            [pltpu.VMEM((CHUNK,), jnp.int32) for _ in range(2)],
            [pltpu.VMEM((CHUNK, 2 * D_MODEL), jnp.float32) for _ in range(2)],
            [pltpu.VMEM((CHUNK, D_MODEL), jnp.float32) for _ in range(2)],
            [pltpu.SemaphoreType.DMA for _ in range(2)],
            [pltpu.SemaphoreType.DMA for _ in range(2)],
        ],
        compiler_params=pltpu.CompilerParams(
            use_tc_tiling_on_sc=True, needs_layout_passes=False
        ),
    )
    def emb(x_hbm, table_hbm, out_hbm, idx_v, rowid, gath, outb, gsem, ssem):
        wid = lax.axis_index("s") * 2 + lax.axis_index("c")
        base = wid * b_per_w
        lane = lax.iota(jnp.int32, 16)

        def start_gather(g, b):
            for i in range(CHUNK // 16):
                sl = pl.ds(g * CHUNK + i * 16, 16)
                rowid[b][pl.ds(i * 16, 16)] = jax.lax.shift_right_logical(
                    idx_v[sl], 1
                )
            pltpu.async_copy(table_hbm.at[rowid[b]], gath[b], gsem[b])

        def wait_gather(b):
            pltpu.make_async_copy(
                table_hbm.at[pl.ds(0, CHUNK)], gath[b], gsem[b]
            ).wait()

        def issue_scatter(g, b):
            dst = out_hbm.at[pl.ds(base + g * CHUNK, CHUNK)]
            pltpu.async_copy(outb[b], dst, ssem[b])

        def wait_scatter(b):
            pltpu.make_async_copy(
                outb[b], out_hbm.at[pl.ds(0, CHUNK)], ssem[b]
            ).wait()

        def turn(g, b, first, last):
            if not first:
                wait_scatter(b)
            wait_gather(b)

            @plsc.parallel_loop(0, CHUNK, step=16)
            def _compact(r0):
                par = (idx_v[pl.ds(g * CHUNK + r0, 16)] & 1) * D_MODEL

                for j in range(16):
                    r = r0 + j
                    rowv = jax.lax.broadcast(r, (16,))
                    src_half = par[j]
                    for c in range(D_MODEL // 16):
                        colv = src_half + c * 16 + lane
                        vals = plsc.load_gather(gath[b], [rowv, colv])
                        outb[b][r, pl.ds(c * 16, 16)] = vals * SCALE

            issue_scatter(g, b)
            if not last:
                start_gather(g + 2, b)

        pltpu.sync_copy(x_hbm.at[pl.ds(base, b_per_w)], idx_v)
        start_gather(0, 0)
        start_gather(1, 1)

        turn(0, 0, True, False)
        turn(1, 1, True, False)

        def cycle(gg, carry):
            turn(2 * gg, 0, False, False)
            turn(2 * gg + 1, 1, False, False)
            return carry

        lax.fori_loop(1, n_chunks // 2 - 1, cycle, 0)

        turn(n_chunks - 2, 0, False, True)
        turn(n_chunks - 1, 1, False, True)

        wait_scatter(0)
        wait_scatter(1)

    out = emb(xf, table2)
    return out.reshape(s0, s1, D_MODEL)
